# Initial kernel scaffold; baseline (speedup 1.0000x reference)
import jax
import jax.numpy as jnp
from jax.experimental import pallas as pl


def kernel(V, E, edges, ew0, eb0, ew1, eb1, ew2, eb2, eg, ebt, nw0, nb0, nw1, nb1, nw2, nb2, ng, nbt):
    raise NotImplementedError("write your pallas kernel here")


# speedup vs baseline: 8075.4653x; 8075.4653x over previous
import functools

import jax
import jax.numpy as jnp
from jax import lax
from jax.experimental import pallas as pl
from jax.experimental.pallas import tpu as pltpu
from jax.experimental.pallas import tpu_sc as plsc

N = 10000
EG = 160000
DN = 256
DE = 128
H = 128
OUT = 256

NC, NS = 2, 16
NW = NC * NS
EPW = EG // NW
CH = 128
NFULL = EPW // CH
TAIL = EPW - NFULL * CH
NPAD = 10240
RPW = NPAD // NS


def _pre_body(v_ref, w_ref, p_ref, q_ref, r_ref):
    y = jnp.dot(v_ref[...], w_ref[...], preferred_element_type=jnp.float32)
    p_ref[...] = y[:, :H]
    q_ref[...] = y[:, H:2 * H]
    r_ref[...] = y[:, 2 * H:3 * H]


def _precompute(v, w_cat):
    blk = 1000
    grid = (N // blk,)
    return pl.pallas_call(
        _pre_body,
        grid=grid,
        in_specs=[
            pl.BlockSpec((blk, DN), lambda i: (i, 0)),
            pl.BlockSpec((DN, 3 * H), lambda i: (0, 0)),
        ],
        out_specs=[
            pl.BlockSpec((blk, H), lambda i: (i, 0)),
            pl.BlockSpec((blk, H), lambda i: (i, 0)),
            pl.BlockSpec((blk, H), lambda i: (i, 0)),
        ],
        out_shape=[
            jax.ShapeDtypeStruct((N, H), jnp.float32),
            jax.ShapeDtypeStruct((N, H), jnp.float32),
            jax.ShapeDtypeStruct((N, H), jnp.float32),
        ],
    )(v, w_cat)


def _mlp_tail(x, w1, b1, w2, b2, g, bt):
    x = jnp.maximum(jnp.dot(x, w1, preferred_element_type=jnp.float32) + b1, 0.0)
    x = jnp.dot(x, w2, preferred_element_type=jnp.float32) + b2
    mu = jnp.mean(x, axis=-1, keepdims=True)
    xc = x - mu
    var = jnp.mean(xc * xc, axis=-1, keepdims=True)
    return xc * lax.rsqrt(var + 1e-5) * g + bt


def _edge_body(ps, qr, e, w0e, b0, w1, b1, w2, b2, g, bt, out):
    x = ps[...] + qr[...] + b0[...]
    x = x + jnp.dot(e[...], w0e[...], preferred_element_type=jnp.float32)
    x = jnp.maximum(x, 0.0)
    out[...] = _mlp_tail(x, w1[...], b1[...], w2[...], b2[...], g[...], bt[...])


def _edge_mlp(ps, qr, e, w0e, b0, w1, b1, w2, b2, g, bt):
    blk = 4000
    grid = (EG // blk,)
    row = lambda i: (i, 0)
    fix = lambda i: (0, 0)
    return pl.pallas_call(
        _edge_body,
        grid=grid,
        in_specs=[
            pl.BlockSpec((blk, H), row),
            pl.BlockSpec((blk, H), row),
            pl.BlockSpec((blk, DE), row),
            pl.BlockSpec((DE, H), fix),
            pl.BlockSpec((1, H), fix),
            pl.BlockSpec((H, H), fix),
            pl.BlockSpec((1, H), fix),
            pl.BlockSpec((H, DE), fix),
            pl.BlockSpec((1, DE), fix),
            pl.BlockSpec((1, DE), fix),
            pl.BlockSpec((1, DE), fix),
        ],
        out_specs=pl.BlockSpec((blk, DE), row),
        out_shape=jax.ShapeDtypeStruct((EG, DE), jnp.float32),
    )(ps, qr, e, w0e, b0, w1, b1, w2, b2, g, bt)


def _node_body(r, p0, p1, w0e, b0, w1, b1, w2, b2, g, bt, out):
    es = p0[...] + p1[...]
    x = r[...] + b0[...]
    x = x + jnp.dot(es, w0e[...], preferred_element_type=jnp.float32)
    x = jnp.maximum(x, 0.0)
    out[...] = _mlp_tail(x, w1[...], b1[...], w2[...], b2[...], g[...], bt[...])


def _node_mlp(r, p0, p1, w0e, b0, w1, b1, w2, b2, g, bt):
    blk = 1000
    grid = (N // blk,)
    row = lambda i: (i, 0)
    fix = lambda i: (0, 0)
    return pl.pallas_call(
        _node_body,
        grid=grid,
        in_specs=[
            pl.BlockSpec((blk, H), row),
            pl.BlockSpec((blk, DE), row),
            pl.BlockSpec((blk, DE), row),
            pl.BlockSpec((DE, H), fix),
            pl.BlockSpec((1, H), fix),
            pl.BlockSpec((H, H), fix),
            pl.BlockSpec((1, H), fix),
            pl.BlockSpec((H, OUT), fix),
            pl.BlockSpec((1, OUT), fix),
            pl.BlockSpec((1, OUT), fix),
            pl.BlockSpec((1, OUT), fix),
        ],
        out_specs=pl.BlockSpec((blk, OUT), row),
        out_shape=jax.ShapeDtypeStruct((N, OUT), jnp.float32),
    )(r, p0, p1, w0e, b0, w1, b1, w2, b2, g, bt)


def _sc_gather_body(p_hbm, q_hbm, idx_s_hbm, idx_r_hbm, ps_out, qr_out,
                    idx_v, rows_v, idx_t, rows_t, sem):
    wid = lax.axis_index("s") * NC + lax.axis_index("c")
    base = wid * EPW

    def one(off, size, idx_hbm, tab, out, ibuf, rbuf):
        pltpu.sync_copy(idx_hbm.at[pl.ds(off, size)], ibuf)
        pltpu.async_copy(tab.at[ibuf], rbuf, sem).wait()
        pltpu.sync_copy(rbuf, out.at[pl.ds(off, size)])

    def body(i, carry):
        off = pl.multiple_of(base + i * CH, 8)
        one(off, CH, idx_s_hbm, p_hbm, ps_out, idx_v, rows_v)
        one(off, CH, idx_r_hbm, q_hbm, qr_out, idx_v, rows_v)
        return carry

    lax.fori_loop(0, NFULL, body, 0)
    toff = pl.multiple_of(base + NFULL * CH, 8)
    one(toff, TAIL, idx_s_hbm, p_hbm, ps_out, idx_t, rows_t)
    one(toff, TAIL, idx_r_hbm, q_hbm, qr_out, idx_t, rows_t)


def _sc_scatter_body(emb_hbm, idx_s_hbm, zeros_hbm, out_hbm,
                     idx_v, rows_v, idx_t, rows_t, acc, sem):
    cid = lax.axis_index("c")
    sid = lax.axis_index("s")
    wid = sid * NC + cid
    pltpu.sync_copy(zeros_hbm, acc.at[pl.ds(sid * RPW, RPW)])
    plsc.subcore_barrier()

    base = wid * EPW

    def one(off, size, ibuf, rbuf):
        pltpu.sync_copy(idx_s_hbm.at[pl.ds(off, size)], ibuf)
        pltpu.sync_copy(emb_hbm.at[pl.ds(off, size)], rbuf)
        pltpu.sync_copy(rbuf, acc.at[ibuf], add=True)

    def body(i, carry):
        off = pl.multiple_of(base + i * CH, 8)
        one(off, CH, idx_v, rows_v)
        return carry

    lax.fori_loop(0, NFULL, body, 0)
    toff = pl.multiple_of(base + NFULL * CH, 8)
    one(toff, TAIL, idx_t, rows_t)

    plsc.subcore_barrier()
    ooff = pl.multiple_of(cid * NPAD + sid * RPW, 8)
    pltpu.sync_copy(acc.at[pl.ds(sid * RPW, RPW)], out_hbm.at[pl.ds(ooff, RPW)])


@functools.lru_cache(maxsize=None)
def _build_sc_kernels():
    mesh = plsc.VectorSubcoreMesh(core_axis_name="c", subcore_axis_name="s",
                                  num_cores=NC, num_subcores=NS)
    gather = pl.kernel(
        _sc_gather_body,
        out_type=(jax.ShapeDtypeStruct((EG, H), jnp.float32),
                  jax.ShapeDtypeStruct((EG, H), jnp.float32)),
        mesh=mesh,
        scratch_types=[
            pltpu.VMEM((CH,), jnp.int32),
            pltpu.VMEM((CH, H), jnp.float32),
            pltpu.VMEM((TAIL,), jnp.int32),
            pltpu.VMEM((TAIL, H), jnp.float32),
            pltpu.SemaphoreType.DMA,
        ],
    )
    scatter = pl.kernel(
        _sc_scatter_body,
        out_type=jax.ShapeDtypeStruct((NC * NPAD, DE), jnp.float32),
        mesh=mesh,
        scratch_types=[
            pltpu.VMEM((CH,), jnp.int32),
            pltpu.VMEM((CH, DE), jnp.float32),
            pltpu.VMEM((TAIL,), jnp.int32),
            pltpu.VMEM((TAIL, DE), jnp.float32),
            pltpu.VMEM_SHARED((NPAD, DE), jnp.float32),
            pltpu.SemaphoreType.DMA,
        ],
    )
    return gather, scatter


def _sc_gather(p, q, idx_s, idx_r):
    return _build_sc_kernels()[0](p, q, idx_s, idx_r)


def _sc_scatter(emb, idx_s, zeros):
    return _build_sc_kernels()[1](emb, idx_s, zeros)


def kernel(V, E, edges, ew0, eb0, ew1, eb1, ew2, eb2, eg, ebt,
           nw0, nb0, nw1, nb1, nw2, nb2, ng, nbt):
    v2 = V.reshape(N, DN)
    e2 = E.reshape(EG, DE)
    idx_s = edges.reshape(EG, 2)[:, 0]
    idx_r = edges.reshape(EG, 2)[:, 1]

    w_cat = jnp.concatenate([ew0[:DN], ew0[DN:2 * DN], nw0[:DN]], axis=1)
    p, q, r = _precompute(v2, w_cat)

    ps, qr = _sc_gather(p, q, idx_s, idx_r)

    edge_emb = _edge_mlp(
        ps, qr, e2, ew0[2 * DN:], eb0.reshape(1, H),
        ew1, eb1.reshape(1, H), ew2, eb2.reshape(1, DE),
        eg.reshape(1, DE), ebt.reshape(1, DE))

    zeros = jnp.zeros((RPW, DE), jnp.float32)
    partials = _sc_scatter(edge_emb, idx_s, zeros)
    p0 = partials[:N]
    p1 = partials[NPAD:NPAD + N]

    node_emb = _node_mlp(
        r, p0, p1, nw0[DN:], nb0.reshape(1, H),
        nw1, nb1.reshape(1, H), nw2, nb2.reshape(1, OUT),
        ng.reshape(1, OUT), nbt.reshape(1, OUT))

    return (node_emb.reshape(1, N, OUT), edge_emb.reshape(1, EG, DE))


# speedup vs baseline: 10372.2505x; 1.2844x over previous
import functools

import jax
import jax.numpy as jnp
from jax import lax
from jax.experimental import pallas as pl
from jax.experimental.pallas import tpu as pltpu
from jax.experimental.pallas import tpu_sc as plsc

N = 10000
EG = 160000
DN = 256
DE = 128
H = 128
OUT = 256

NC, NS = 2, 16
NW = NC * NS
EPW = EG // NW
CH = 128
NFULL = EPW // CH
TAIL = EPW - NFULL * CH
NPAD = 10240
RPW = NPAD // NS


def _pre_body(v_ref, w_ref, p_ref, q_ref, r_ref):
    y = jnp.dot(v_ref[...], w_ref[...], preferred_element_type=jnp.float32)
    p_ref[...] = y[:, :H]
    q_ref[...] = y[:, H:2 * H]
    r_ref[...] = y[:, 2 * H:3 * H]


def _precompute(v, w_cat):
    blk = 1000
    grid = (N // blk,)
    row = lambda i: (i, 0)
    fix = lambda i: (0, 0)
    return pl.pallas_call(
        _pre_body,
        grid=grid,
        in_specs=[
            pl.BlockSpec((blk, DN), row),
            pl.BlockSpec((DN, 3 * H), fix),
        ],
        out_specs=[
            pl.BlockSpec((blk, H), row),
            pl.BlockSpec((blk, H), row),
            pl.BlockSpec((blk, H), row),
        ],
        out_shape=[
            jax.ShapeDtypeStruct((N, H), jnp.float32),
            jax.ShapeDtypeStruct((N, H), jnp.float32),
            jax.ShapeDtypeStruct((N, H), jnp.float32),
        ],
    )(v, w_cat)


def _edge_body(ps, qr, e, w0e, b0, w1, b1, w2, b2, g, bt, out):
    x = ps[...] + qr[...] + b0[...]
    x = x + jnp.dot(e[...], w0e[...], preferred_element_type=jnp.float32)
    x = jnp.maximum(x, 0.0)
    x = jnp.dot(x.astype(jnp.bfloat16), w1[...],
                preferred_element_type=jnp.float32) + b1[...]
    x = jnp.maximum(x, 0.0)
    x = jnp.dot(x.astype(jnp.bfloat16), w2[...],
                preferred_element_type=jnp.float32) + b2[...]
    mu = jnp.mean(x, axis=-1, keepdims=True)
    xc = x - mu
    var = jnp.mean(xc * xc, axis=-1, keepdims=True)
    out[...] = xc * lax.rsqrt(var + 1e-5) * g[...] + bt[...]


def _edge_mlp(ps, qr, e, w0e, b0, w1, b1, w2, b2, g, bt):
    blk = 4000
    grid = (EG // blk,)
    row = lambda i: (i, 0)
    fix = lambda i: (0, 0)
    return pl.pallas_call(
        _edge_body,
        grid=grid,
        in_specs=[
            pl.BlockSpec((blk, H), row),
            pl.BlockSpec((blk, H), row),
            pl.BlockSpec((blk, DE), row),
            pl.BlockSpec((DE, H), fix),
            pl.BlockSpec((1, H), fix),
            pl.BlockSpec((H, H), fix),
            pl.BlockSpec((1, H), fix),
            pl.BlockSpec((H, DE), fix),
            pl.BlockSpec((1, DE), fix),
            pl.BlockSpec((1, DE), fix),
            pl.BlockSpec((1, DE), fix),
        ],
        out_specs=pl.BlockSpec((blk, DE), row),
        out_shape=jax.ShapeDtypeStruct((EG, DE), jnp.float32),
    )(ps, qr, e, w0e, b0, w1, b1, w2, b2, g, bt)


def _node_body(r, p0, p1, w0e, b0, w1, b1, w2, b2, g, bt, out):
    es = p0[...] + p1[...]
    x = r[...] + b0[...]
    x = x + jnp.dot(es, w0e[...], preferred_element_type=jnp.float32)
    x = jnp.maximum(x, 0.0)
    x = jnp.maximum(jnp.dot(x, w1[...], preferred_element_type=jnp.float32)
                    + b1[...], 0.0)
    x = jnp.dot(x, w2[...], preferred_element_type=jnp.float32) + b2[...]
    mu = jnp.mean(x, axis=-1, keepdims=True)
    xc = x - mu
    var = jnp.mean(xc * xc, axis=-1, keepdims=True)
    out[...] = xc * lax.rsqrt(var + 1e-5) * g[...] + bt[...]


def _node_mlp(r, p0, p1, w0e, b0, w1, b1, w2, b2, g, bt):
    blk = 1000
    grid = (N // blk,)
    row = lambda i: (i, 0)
    fix = lambda i: (0, 0)
    return pl.pallas_call(
        _node_body,
        grid=grid,
        in_specs=[
            pl.BlockSpec((blk, H), row),
            pl.BlockSpec((blk, DE), row),
            pl.BlockSpec((blk, DE), row),
            pl.BlockSpec((DE, H), fix),
            pl.BlockSpec((1, H), fix),
            pl.BlockSpec((H, H), fix),
            pl.BlockSpec((1, H), fix),
            pl.BlockSpec((H, OUT), fix),
            pl.BlockSpec((1, OUT), fix),
            pl.BlockSpec((1, OUT), fix),
            pl.BlockSpec((1, OUT), fix),
        ],
        out_specs=pl.BlockSpec((blk, OUT), row),
        out_shape=jax.ShapeDtypeStruct((N, OUT), jnp.float32),
    )(r, p0, p1, w0e, b0, w1, b1, w2, b2, g, bt)


def _sc_gather_body(p_hbm, q_hbm, idx_s_hbm, idx_r_hbm, ps_out, qr_out,
                    isv, irv, pb, qb, ptb, qtb, semA, semB, semC, semD):
    wid = lax.axis_index("s") * NC + lax.axis_index("c")
    base = pl.multiple_of(wid * EPW, 8)
    h1 = pltpu.async_copy(idx_s_hbm.at[pl.ds(base, EPW)], isv, semA)
    h2 = pltpu.async_copy(idx_r_hbm.at[pl.ds(base, EPW)], irv, semB)
    h1.wait()
    h2.wait()

    def body(i, carry):
        off = pl.multiple_of(i * CH, 8)
        goff = pl.multiple_of(base + i * CH, 8)
        g1 = pltpu.async_copy(p_hbm.at[isv.at[pl.ds(off, CH)]], pb, semA)
        g2 = pltpu.async_copy(q_hbm.at[irv.at[pl.ds(off, CH)]], qb, semB)
        g1.wait()
        w1 = pltpu.async_copy(pb, ps_out.at[pl.ds(goff, CH)], semC)
        g2.wait()
        w2 = pltpu.async_copy(qb, qr_out.at[pl.ds(goff, CH)], semD)
        w1.wait()
        w2.wait()
        return carry

    lax.fori_loop(0, NFULL, body, 0)

    toff = pl.multiple_of(NFULL * CH, 8)
    gtoff = pl.multiple_of(base + NFULL * CH, 8)
    g1 = pltpu.async_copy(p_hbm.at[isv.at[pl.ds(toff, TAIL)]], ptb, semA)
    g2 = pltpu.async_copy(q_hbm.at[irv.at[pl.ds(toff, TAIL)]], qtb, semB)
    g1.wait()
    w1 = pltpu.async_copy(ptb, ps_out.at[pl.ds(gtoff, TAIL)], semC)
    g2.wait()
    w2 = pltpu.async_copy(qtb, qr_out.at[pl.ds(gtoff, TAIL)], semD)
    w1.wait()
    w2.wait()


def _sc_scatter_body(emb_hbm, idx_s_hbm, zeros_hbm, out_hbm,
                     ib0, rb0, ib1, rb1, it, rt, acc,
                     sI0, sE0, sI1, sE1):
    cid = lax.axis_index("c")
    sid = lax.axis_index("s")
    wid = sid * NC + cid
    pltpu.sync_copy(zeros_hbm, acc.at[pl.ds(sid * RPW, RPW)])
    plsc.subcore_barrier()

    base = pl.multiple_of(wid * EPW, 8)

    def pre(i, ib, rb, sI, sE):
        off = pl.multiple_of(base + i * CH, 8)
        pltpu.async_copy(idx_s_hbm.at[pl.ds(off, CH)], ib, sI)
        pltpu.async_copy(emb_hbm.at[pl.ds(off, CH)], rb, sE)

    def wait_pre(ib, rb, sI, sE):
        pltpu.make_async_copy(idx_s_hbm.at[pl.ds(0, CH)], ib, sI).wait()
        pltpu.make_async_copy(emb_hbm.at[pl.ds(0, CH)], rb, sE).wait()

    def scat(ib, rb):
        pltpu.sync_copy(rb, acc.at[ib], add=True)

    pre(0, ib0, rb0, sI0, sE0)
    pre(1, ib1, rb1, sI1, sE1)

    def body(k, carry):
        i = 2 * k
        wait_pre(ib0, rb0, sI0, sE0)
        scat(ib0, rb0)
        pre(i + 2, ib0, rb0, sI0, sE0)
        wait_pre(ib1, rb1, sI1, sE1)
        scat(ib1, rb1)
        pre(i + 3, ib1, rb1, sI1, sE1)
        return carry

    lax.fori_loop(0, 18, body, 0)
    wait_pre(ib0, rb0, sI0, sE0)
    scat(ib0, rb0)
    pre(38, ib0, rb0, sI0, sE0)
    wait_pre(ib1, rb1, sI1, sE1)
    scat(ib1, rb1)
    toff = pl.multiple_of(base + NFULL * CH, 8)
    pltpu.async_copy(idx_s_hbm.at[pl.ds(toff, TAIL)], it, sI1)
    pltpu.async_copy(emb_hbm.at[pl.ds(toff, TAIL)], rt, sE1)
    wait_pre(ib0, rb0, sI0, sE0)
    scat(ib0, rb0)
    pltpu.make_async_copy(idx_s_hbm.at[pl.ds(0, TAIL)], it, sI1).wait()
    pltpu.make_async_copy(emb_hbm.at[pl.ds(0, TAIL)], rt, sE1).wait()
    pltpu.sync_copy(rt, acc.at[it], add=True)

    plsc.subcore_barrier()
    ooff = pl.multiple_of(cid * NPAD + sid * RPW, 8)
    pltpu.sync_copy(acc.at[pl.ds(sid * RPW, RPW)], out_hbm.at[pl.ds(ooff, RPW)])


@functools.lru_cache(maxsize=None)
def _build_sc_kernels():
    mesh = plsc.VectorSubcoreMesh(core_axis_name="c", subcore_axis_name="s",
                                  num_cores=NC, num_subcores=NS)
    gather = pl.kernel(
        _sc_gather_body,
        out_type=(jax.ShapeDtypeStruct((EG, H), jnp.float32),
                  jax.ShapeDtypeStruct((EG, H), jnp.float32)),
        mesh=mesh,
        scratch_types=[
            pltpu.VMEM((EPW,), jnp.int32),
            pltpu.VMEM((EPW,), jnp.int32),
            pltpu.VMEM((CH, H), jnp.float32),
            pltpu.VMEM((CH, H), jnp.float32),
            pltpu.VMEM((TAIL, H), jnp.float32),
            pltpu.VMEM((TAIL, H), jnp.float32),
            pltpu.SemaphoreType.DMA,
            pltpu.SemaphoreType.DMA,
            pltpu.SemaphoreType.DMA,
            pltpu.SemaphoreType.DMA,
        ],
    )
    scatter = pl.kernel(
        _sc_scatter_body,
        out_type=jax.ShapeDtypeStruct((NC * NPAD, DE), jnp.float32),
        mesh=mesh,
        scratch_types=[
            pltpu.VMEM((CH,), jnp.int32),
            pltpu.VMEM((CH, DE), jnp.float32),
            pltpu.VMEM((CH,), jnp.int32),
            pltpu.VMEM((CH, DE), jnp.float32),
            pltpu.VMEM((TAIL,), jnp.int32),
            pltpu.VMEM((TAIL, DE), jnp.float32),
            pltpu.VMEM_SHARED((NPAD, DE), jnp.float32),
            pltpu.SemaphoreType.DMA,
            pltpu.SemaphoreType.DMA,
            pltpu.SemaphoreType.DMA,
            pltpu.SemaphoreType.DMA,
        ],
    )
    return gather, scatter


def _sc_gather(p, q, idx_s, idx_r):
    return _build_sc_kernels()[0](p, q, idx_s, idx_r)


def _sc_scatter(emb, idx_s, zeros):
    return _build_sc_kernels()[1](emb, idx_s, zeros)


def kernel(V, E, edges, ew0, eb0, ew1, eb1, ew2, eb2, eg, ebt,
           nw0, nb0, nw1, nb1, nw2, nb2, ng, nbt):
    v2 = V.reshape(N, DN)
    e2 = E.reshape(EG, DE).astype(jnp.bfloat16)
    idx_s = edges.reshape(EG, 2)[:, 0]
    idx_r = edges.reshape(EG, 2)[:, 1]

    w_cat = jnp.concatenate([ew0[:DN], ew0[DN:2 * DN], nw0[:DN]], axis=1)
    p, q, r = _precompute(v2, w_cat)

    ps, qr = _sc_gather(p, q, idx_s, idx_r)

    edge_emb = _edge_mlp(
        ps, qr, e2, ew0[2 * DN:].astype(jnp.bfloat16), eb0.reshape(1, H),
        ew1.astype(jnp.bfloat16), eb1.reshape(1, H),
        ew2.astype(jnp.bfloat16), eb2.reshape(1, DE),
        eg.reshape(1, DE), ebt.reshape(1, DE))

    zeros = jnp.zeros((RPW, DE), jnp.float32)
    partials = _sc_scatter(edge_emb, idx_s, zeros)
    p0 = partials[:N]
    p1 = partials[NPAD:NPAD + N]

    node_emb = _node_mlp(
        r, p0, p1, nw0[DN:], nb0.reshape(1, H),
        nw1, nb1.reshape(1, H), nw2, nb2.reshape(1, OUT),
        ng.reshape(1, OUT), nbt.reshape(1, OUT))

    return (node_emb.reshape(1, N, OUT), edge_emb.reshape(1, EG, DE))


# speedup vs baseline: 10538.0573x; 1.0160x over previous
import functools

import jax
import jax.numpy as jnp
from jax import lax
from jax.experimental import pallas as pl
from jax.experimental.pallas import tpu as pltpu
from jax.experimental.pallas import tpu_sc as plsc

N = 10000
EG = 160000
DN = 256
DE = 128
H = 128
OUT = 256

NC, NS = 2, 16
NW = NC * NS
EPW = EG // NW
CH = 128
NFULL = EPW // CH
TAIL = EPW - NFULL * CH
NPAD = 10240
RPW = NPAD // NS


def _pre_body(v_ref, w_ref, p_ref, q_ref, r_ref):
    y = jnp.dot(v_ref[...], w_ref[...], preferred_element_type=jnp.float32)
    p_ref[...] = y[:, :H]
    q_ref[...] = y[:, H:2 * H]
    r_ref[...] = y[:, 2 * H:3 * H]


def _precompute(v, w_cat):
    blk = 1000
    grid = (N // blk,)
    row = lambda i: (i, 0)
    fix = lambda i: (0, 0)
    return pl.pallas_call(
        _pre_body,
        grid=grid,
        in_specs=[
            pl.BlockSpec((blk, DN), row),
            pl.BlockSpec((DN, 3 * H), fix),
        ],
        out_specs=[
            pl.BlockSpec((blk, H), row),
            pl.BlockSpec((blk, H), row),
            pl.BlockSpec((blk, H), row),
        ],
        out_shape=[
            jax.ShapeDtypeStruct((N, H), jnp.float32),
            jax.ShapeDtypeStruct((N, H), jnp.float32),
            jax.ShapeDtypeStruct((N, H), jnp.float32),
        ],
    )(v, w_cat)


def _edge_body(pq, e, w0e, b0, w1, b1, w2, b2, g, bt, out):
    x = pq[...] + b0[...]
    x = x + jnp.dot(e[...], w0e[...], preferred_element_type=jnp.float32)
    x = jnp.maximum(x, 0.0)
    x = jnp.dot(x.astype(jnp.bfloat16), w1[...],
                preferred_element_type=jnp.float32) + b1[...]
    x = jnp.maximum(x, 0.0)
    x = jnp.dot(x.astype(jnp.bfloat16), w2[...],
                preferred_element_type=jnp.float32) + b2[...]
    mu = jnp.mean(x, axis=-1, keepdims=True)
    xc = x - mu
    var = jnp.mean(xc * xc, axis=-1, keepdims=True)
    out[...] = xc * lax.rsqrt(var + 1e-5) * g[...] + bt[...]


def _edge_mlp(pq, e, w0e, b0, w1, b1, w2, b2, g, bt):
    blk = 4000
    grid = (EG // blk,)
    row = lambda i: (i, 0)
    fix = lambda i: (0, 0)
    return pl.pallas_call(
        _edge_body,
        grid=grid,
        in_specs=[
            pl.BlockSpec((blk, H), row),
            pl.BlockSpec((blk, DE), row),
            pl.BlockSpec((DE, H), fix),
            pl.BlockSpec((1, H), fix),
            pl.BlockSpec((H, H), fix),
            pl.BlockSpec((1, H), fix),
            pl.BlockSpec((H, DE), fix),
            pl.BlockSpec((1, DE), fix),
            pl.BlockSpec((1, DE), fix),
            pl.BlockSpec((1, DE), fix),
        ],
        out_specs=pl.BlockSpec((blk, DE), row),
        out_shape=jax.ShapeDtypeStruct((EG, DE), jnp.float32),
    )(pq, e, w0e, b0, w1, b1, w2, b2, g, bt)


def _node_body(r, p0, p1, w0e, b0, w1, b1, w2, b2, g, bt, out):
    es = p0[...] + p1[...]
    x = r[...] + b0[...]
    x = x + jnp.dot(es, w0e[...], preferred_element_type=jnp.float32)
    x = jnp.maximum(x, 0.0)
    x = jnp.maximum(jnp.dot(x, w1[...], preferred_element_type=jnp.float32)
                    + b1[...], 0.0)
    x = jnp.dot(x, w2[...], preferred_element_type=jnp.float32) + b2[...]
    mu = jnp.mean(x, axis=-1, keepdims=True)
    xc = x - mu
    var = jnp.mean(xc * xc, axis=-1, keepdims=True)
    out[...] = xc * lax.rsqrt(var + 1e-5) * g[...] + bt[...]


def _node_mlp(r, p0, p1, w0e, b0, w1, b1, w2, b2, g, bt):
    blk = 1000
    grid = (N // blk,)
    row = lambda i: (i, 0)
    fix = lambda i: (0, 0)
    return pl.pallas_call(
        _node_body,
        grid=grid,
        in_specs=[
            pl.BlockSpec((blk, H), row),
            pl.BlockSpec((blk, DE), row),
            pl.BlockSpec((blk, DE), row),
            pl.BlockSpec((DE, H), fix),
            pl.BlockSpec((1, H), fix),
            pl.BlockSpec((H, H), fix),
            pl.BlockSpec((1, H), fix),
            pl.BlockSpec((H, OUT), fix),
            pl.BlockSpec((1, OUT), fix),
            pl.BlockSpec((1, OUT), fix),
            pl.BlockSpec((1, OUT), fix),
        ],
        out_specs=pl.BlockSpec((blk, OUT), row),
        out_shape=jax.ShapeDtypeStruct((N, OUT), jnp.float32),
    )(r, p0, p1, w0e, b0, w1, b1, w2, b2, g, bt)


def _sc_gather_body(p_hbm, q_hbm, idx_s_hbm, idx_r_hbm, pq_out,
                    isv, irv, pb, ptb, semA, semB, semC):
    wid = lax.axis_index("s") * NC + lax.axis_index("c")
    base = pl.multiple_of(wid * EPW, 8)
    h1 = pltpu.async_copy(idx_s_hbm.at[pl.ds(base, EPW)], isv, semA)
    h2 = pltpu.async_copy(idx_r_hbm.at[pl.ds(base, EPW)], irv, semB)
    h1.wait()
    h2.wait()

    def body(i, carry):
        off = pl.multiple_of(i * CH, 8)
        goff = pl.multiple_of(base + i * CH, 8)
        g1 = pltpu.async_copy(p_hbm.at[isv.at[pl.ds(off, CH)]], pb, semA)
        g1.wait()
        g2 = pltpu.async_copy(q_hbm.at[irv.at[pl.ds(off, CH)]], pb, semB,
                              add=True)
        g2.wait()
        w1 = pltpu.async_copy(pb, pq_out.at[pl.ds(goff, CH)], semC)
        w1.wait()
        return carry

    lax.fori_loop(0, NFULL, body, 0)

    toff = pl.multiple_of(NFULL * CH, 8)
    gtoff = pl.multiple_of(base + NFULL * CH, 8)
    g1 = pltpu.async_copy(p_hbm.at[isv.at[pl.ds(toff, TAIL)]], ptb, semA)
    g1.wait()
    g2 = pltpu.async_copy(q_hbm.at[irv.at[pl.ds(toff, TAIL)]], ptb, semB,
                          add=True)
    g2.wait()
    w1 = pltpu.async_copy(ptb, pq_out.at[pl.ds(gtoff, TAIL)], semC)
    w1.wait()


def _sc_scatter_body(emb_hbm, idx_s_hbm, zeros_hbm, out_hbm,
                     ib0, rb0, ib1, rb1, it, rt, acc,
                     sI0, sE0, sI1, sE1):
    cid = lax.axis_index("c")
    sid = lax.axis_index("s")
    wid = sid * NC + cid
    pltpu.sync_copy(zeros_hbm, acc.at[pl.ds(sid * RPW, RPW)])
    plsc.subcore_barrier()

    base = pl.multiple_of(wid * EPW, 8)

    def pre(i, ib, rb, sI, sE):
        off = pl.multiple_of(base + i * CH, 8)
        pltpu.async_copy(idx_s_hbm.at[pl.ds(off, CH)], ib, sI)
        pltpu.async_copy(emb_hbm.at[pl.ds(off, CH)], rb, sE)

    def wait_pre(ib, rb, sI, sE):
        pltpu.make_async_copy(idx_s_hbm.at[pl.ds(0, CH)], ib, sI).wait()
        pltpu.make_async_copy(emb_hbm.at[pl.ds(0, CH)], rb, sE).wait()

    def scat(ib, rb):
        pltpu.sync_copy(rb, acc.at[ib], add=True)

    pre(0, ib0, rb0, sI0, sE0)
    pre(1, ib1, rb1, sI1, sE1)

    def body(k, carry):
        i = 2 * k
        wait_pre(ib0, rb0, sI0, sE0)
        scat(ib0, rb0)
        pre(i + 2, ib0, rb0, sI0, sE0)
        wait_pre(ib1, rb1, sI1, sE1)
        scat(ib1, rb1)
        pre(i + 3, ib1, rb1, sI1, sE1)
        return carry

    lax.fori_loop(0, 18, body, 0)
    wait_pre(ib0, rb0, sI0, sE0)
    scat(ib0, rb0)
    pre(38, ib0, rb0, sI0, sE0)
    wait_pre(ib1, rb1, sI1, sE1)
    scat(ib1, rb1)
    toff = pl.multiple_of(base + NFULL * CH, 8)
    pltpu.async_copy(idx_s_hbm.at[pl.ds(toff, TAIL)], it, sI1)
    pltpu.async_copy(emb_hbm.at[pl.ds(toff, TAIL)], rt, sE1)
    wait_pre(ib0, rb0, sI0, sE0)
    scat(ib0, rb0)
    pltpu.make_async_copy(idx_s_hbm.at[pl.ds(0, TAIL)], it, sI1).wait()
    pltpu.make_async_copy(emb_hbm.at[pl.ds(0, TAIL)], rt, sE1).wait()
    pltpu.sync_copy(rt, acc.at[it], add=True)

    plsc.subcore_barrier()
    ooff = pl.multiple_of(cid * NPAD + sid * RPW, 8)
    pltpu.sync_copy(acc.at[pl.ds(sid * RPW, RPW)], out_hbm.at[pl.ds(ooff, RPW)])


@functools.lru_cache(maxsize=None)
def _build_sc_kernels():
    mesh = plsc.VectorSubcoreMesh(core_axis_name="c", subcore_axis_name="s",
                                  num_cores=NC, num_subcores=NS)
    gather = pl.kernel(
        _sc_gather_body,
        out_type=jax.ShapeDtypeStruct((EG, H), jnp.float32),
        mesh=mesh,
        scratch_types=[
            pltpu.VMEM((EPW,), jnp.int32),
            pltpu.VMEM((EPW,), jnp.int32),
            pltpu.VMEM((CH, H), jnp.float32),
            pltpu.VMEM((TAIL, H), jnp.float32),
            pltpu.SemaphoreType.DMA,
            pltpu.SemaphoreType.DMA,
            pltpu.SemaphoreType.DMA,
        ],
    )
    scatter = pl.kernel(
        _sc_scatter_body,
        out_type=jax.ShapeDtypeStruct((NC * NPAD, DE), jnp.float32),
        mesh=mesh,
        scratch_types=[
            pltpu.VMEM((CH,), jnp.int32),
            pltpu.VMEM((CH, DE), jnp.float32),
            pltpu.VMEM((CH,), jnp.int32),
            pltpu.VMEM((CH, DE), jnp.float32),
            pltpu.VMEM((TAIL,), jnp.int32),
            pltpu.VMEM((TAIL, DE), jnp.float32),
            pltpu.VMEM_SHARED((NPAD, DE), jnp.float32),
            pltpu.SemaphoreType.DMA,
            pltpu.SemaphoreType.DMA,
            pltpu.SemaphoreType.DMA,
            pltpu.SemaphoreType.DMA,
        ],
    )
    return gather, scatter


def _sc_gather(p, q, idx_s, idx_r):
    return _build_sc_kernels()[0](p, q, idx_s, idx_r)


def _sc_scatter(emb, idx_s, zeros):
    return _build_sc_kernels()[1](emb, idx_s, zeros)


def kernel(V, E, edges, ew0, eb0, ew1, eb1, ew2, eb2, eg, ebt,
           nw0, nb0, nw1, nb1, nw2, nb2, ng, nbt):
    v2 = V.reshape(N, DN)
    e2 = E.reshape(EG, DE).astype(jnp.bfloat16)
    idx_s = edges.reshape(EG, 2)[:, 0]
    idx_r = edges.reshape(EG, 2)[:, 1]

    w_cat = jnp.concatenate([ew0[:DN], ew0[DN:2 * DN], nw0[:DN]], axis=1)
    p, q, r = _precompute(v2, w_cat)

    pq = _sc_gather(p, q, idx_s, idx_r)

    edge_emb = _edge_mlp(
        pq, e2, ew0[2 * DN:].astype(jnp.bfloat16), eb0.reshape(1, H),
        ew1.astype(jnp.bfloat16), eb1.reshape(1, H),
        ew2.astype(jnp.bfloat16), eb2.reshape(1, DE),
        eg.reshape(1, DE), ebt.reshape(1, DE))

    zeros = jnp.zeros((RPW, DE), jnp.float32)
    partials = _sc_scatter(edge_emb, idx_s, zeros)
    p0 = partials[:N]
    p1 = partials[NPAD:NPAD + N]

    node_emb = _node_mlp(
        r, p0, p1, nw0[DN:], nb0.reshape(1, H),
        nw1, nb1.reshape(1, H), nw2, nb2.reshape(1, OUT),
        ng.reshape(1, OUT), nbt.reshape(1, OUT))

    return (node_emb.reshape(1, N, OUT), edge_emb.reshape(1, EG, DE))


# speedup vs baseline: 11603.1509x; 1.1011x over previous
import functools

import jax
import jax.numpy as jnp
from jax import lax
from jax.experimental import pallas as pl
from jax.experimental.pallas import tpu as pltpu
from jax.experimental.pallas import tpu_sc as plsc

N = 10000
EG = 160000
DN = 256
DE = 128
H = 128
OUT = 256

NC, NS = 2, 16
NW = NC * NS
EPW = EG // NW
CH = 128
NFULL = EPW // CH
TAIL = EPW - NFULL * CH
NPAD = 10240
RPW = NPAD // NS


def _pre_body(v_ref, w_ref, p_ref, q_ref, r_ref):
    y = jnp.dot(v_ref[...], w_ref[...], preferred_element_type=jnp.float32)
    p_ref[...] = y[:, :H]
    q_ref[...] = y[:, H:2 * H]
    r_ref[...] = y[:, 2 * H:3 * H]


def _precompute(v, w_cat):
    blk = 1000
    grid = (N // blk,)
    row = lambda i: (i, 0)
    fix = lambda i: (0, 0)
    return pl.pallas_call(
        _pre_body,
        grid=grid,
        in_specs=[
            pl.BlockSpec((blk, DN), row),
            pl.BlockSpec((DN, 3 * H), fix),
        ],
        out_specs=[
            pl.BlockSpec((blk, H), row),
            pl.BlockSpec((blk, H), row),
            pl.BlockSpec((blk, H), row),
        ],
        out_shape=[
            jax.ShapeDtypeStruct((N, H), jnp.float32),
            jax.ShapeDtypeStruct((N, H), jnp.float32),
            jax.ShapeDtypeStruct((N, H), jnp.float32),
        ],
    )(v, w_cat)


def _edge_body(pq, e, w0e, b0, w1, b1, w2, b2, g, bt, out):
    x = pq[...] + b0[...]
    x = x + jnp.dot(e[...], w0e[...], preferred_element_type=jnp.float32)
    x = jnp.maximum(x, 0.0)
    x = jnp.dot(x.astype(jnp.bfloat16), w1[...],
                preferred_element_type=jnp.float32) + b1[...]
    x = jnp.maximum(x, 0.0)
    x = jnp.dot(x.astype(jnp.bfloat16), w2[...],
                preferred_element_type=jnp.float32) + b2[...]
    mu = jnp.mean(x, axis=-1, keepdims=True)
    xc = x - mu
    var = jnp.mean(xc * xc, axis=-1, keepdims=True)
    out[...] = xc * lax.rsqrt(var + 1e-5) * g[...] + bt[...]


def _edge_mlp(pq, e, w0e, b0, w1, b1, w2, b2, g, bt):
    blk = 4000
    grid = (EG // blk,)
    row = lambda i: (i, 0)
    fix = lambda i: (0, 0)
    return pl.pallas_call(
        _edge_body,
        grid=grid,
        in_specs=[
            pl.BlockSpec((blk, H), row),
            pl.BlockSpec((blk, DE), row),
            pl.BlockSpec((DE, H), fix),
            pl.BlockSpec((1, H), fix),
            pl.BlockSpec((H, H), fix),
            pl.BlockSpec((1, H), fix),
            pl.BlockSpec((H, DE), fix),
            pl.BlockSpec((1, DE), fix),
            pl.BlockSpec((1, DE), fix),
            pl.BlockSpec((1, DE), fix),
        ],
        out_specs=pl.BlockSpec((blk, DE), row),
        out_shape=jax.ShapeDtypeStruct((EG, DE), jnp.float32),
    )(pq, e, w0e, b0, w1, b1, w2, b2, g, bt)


def _node_body(r, p0, p1, w0e, b0, w1, b1, w2, b2, g, bt, out):
    es = p0[...] + p1[...]
    x = r[...] + b0[...]
    x = x + jnp.dot(es, w0e[...], preferred_element_type=jnp.float32)
    x = jnp.maximum(x, 0.0)
    x = jnp.maximum(jnp.dot(x, w1[...], preferred_element_type=jnp.float32)
                    + b1[...], 0.0)
    x = jnp.dot(x, w2[...], preferred_element_type=jnp.float32) + b2[...]
    mu = jnp.mean(x, axis=-1, keepdims=True)
    xc = x - mu
    var = jnp.mean(xc * xc, axis=-1, keepdims=True)
    out[...] = xc * lax.rsqrt(var + 1e-5) * g[...] + bt[...]


def _node_mlp(r, p0, p1, w0e, b0, w1, b1, w2, b2, g, bt):
    blk = 1000
    grid = (N // blk,)
    row = lambda i: (i, 0)
    fix = lambda i: (0, 0)
    return pl.pallas_call(
        _node_body,
        grid=grid,
        in_specs=[
            pl.BlockSpec((blk, H), row),
            pl.BlockSpec((blk, DE), row),
            pl.BlockSpec((blk, DE), row),
            pl.BlockSpec((DE, H), fix),
            pl.BlockSpec((1, H), fix),
            pl.BlockSpec((H, H), fix),
            pl.BlockSpec((1, H), fix),
            pl.BlockSpec((H, OUT), fix),
            pl.BlockSpec((1, OUT), fix),
            pl.BlockSpec((1, OUT), fix),
            pl.BlockSpec((1, OUT), fix),
        ],
        out_specs=pl.BlockSpec((blk, OUT), row),
        out_shape=jax.ShapeDtypeStruct((N, OUT), jnp.float32),
    )(r, p0, p1, w0e, b0, w1, b1, w2, b2, g, bt)


def _sc_gather_body(p_hbm, q_hbm, idx_s_hbm, idx_r_hbm, pq_out,
                    isv, irv, pb0, pb1, ptb,
                    semA0, semB0, semC0, semA1, semB1, semC1):
    wid = lax.axis_index("s") * NC + lax.axis_index("c")
    base = pl.multiple_of(wid * EPW, 8)
    h1 = pltpu.async_copy(idx_s_hbm.at[pl.ds(base, EPW)], isv, semA0)
    h2 = pltpu.async_copy(idx_r_hbm.at[pl.ds(base, EPW)], irv, semB0)
    h1.wait()
    h2.wait()

    def startP(i, rb, sem, size=CH):
        off = pl.multiple_of(i * CH, 8)
        return pltpu.async_copy(p_hbm.at[isv.at[pl.ds(off, size)]], rb, sem)

    def startQ(i, rb, sem, size=CH):
        off = pl.multiple_of(i * CH, 8)
        return pltpu.async_copy(q_hbm.at[irv.at[pl.ds(off, size)]], rb, sem,
                                add=True)

    def startW(i, rb, sem, size=CH):
        goff = pl.multiple_of(base + i * CH, 8)
        return pltpu.async_copy(rb, pq_out.at[pl.ds(goff, size)], sem)

    def drainW(rb, sem):
        pltpu.make_async_copy(pq_out.at[pl.ds(0, CH)], rb, sem).wait()

    hP0 = startP(0, pb0, semA0)
    hP1 = startP(1, pb1, semA1)
    hP0.wait()
    hQ0 = startQ(0, pb0, semB0)
    hP1.wait()
    hQ1 = startQ(1, pb1, semB1)
    hQ0.wait()
    startW(0, pb0, semC0)
    hQ1.wait()
    startW(1, pb1, semC1)

    def body(k, carry):
        i = 2 * k
        drainW(pb0, semC0)
        hP0 = startP(i, pb0, semA0)
        drainW(pb1, semC1)
        hP1 = startP(i + 1, pb1, semA1)
        hP0.wait()
        hQ0 = startQ(i, pb0, semB0)
        hP1.wait()
        hQ1 = startQ(i + 1, pb1, semB1)
        hQ0.wait()
        startW(i, pb0, semC0)
        hQ1.wait()
        startW(i + 1, pb1, semC1)
        return carry

    lax.fori_loop(1, (NFULL - 1) // 2, body, 0)

    drainW(pb0, semC0)
    hP38 = startP(NFULL - 1, pb0, semA0)
    drainW(pb1, semC1)
    hPt = startP(NFULL, ptb, semA1, size=TAIL)
    hP38.wait()
    hQ38 = startQ(NFULL - 1, pb0, semB0)
    hPt.wait()
    hQt = startQ(NFULL, ptb, semB1, size=TAIL)
    hQ38.wait()
    hW38 = startW(NFULL - 1, pb0, semC0)
    hQt.wait()
    hWt = startW(NFULL, ptb, semC1, size=TAIL)
    hW38.wait()
    hWt.wait()


def _sc_scatter_body(emb_hbm, idx_s_hbm, zeros_hbm, out_hbm,
                     ib0, rb0, ib1, rb1, it, rt, acc,
                     sI0, sE0, sI1, sE1):
    cid = lax.axis_index("c")
    sid = lax.axis_index("s")
    wid = sid * NC + cid
    pltpu.sync_copy(zeros_hbm, acc.at[pl.ds(sid * RPW, RPW)])
    plsc.subcore_barrier()

    base = pl.multiple_of(wid * EPW, 8)

    def pre(i, ib, rb, sI, sE):
        off = pl.multiple_of(base + i * CH, 8)
        pltpu.async_copy(idx_s_hbm.at[pl.ds(off, CH)], ib, sI)
        pltpu.async_copy(emb_hbm.at[pl.ds(off, CH)], rb, sE)

    def wait_pre(ib, rb, sI, sE):
        pltpu.make_async_copy(idx_s_hbm.at[pl.ds(0, CH)], ib, sI).wait()
        pltpu.make_async_copy(emb_hbm.at[pl.ds(0, CH)], rb, sE).wait()

    def scat(ib, rb):
        pltpu.sync_copy(rb, acc.at[ib], add=True)

    pre(0, ib0, rb0, sI0, sE0)
    pre(1, ib1, rb1, sI1, sE1)

    def body(k, carry):
        i = 2 * k
        wait_pre(ib0, rb0, sI0, sE0)
        scat(ib0, rb0)
        pre(i + 2, ib0, rb0, sI0, sE0)
        wait_pre(ib1, rb1, sI1, sE1)
        scat(ib1, rb1)
        pre(i + 3, ib1, rb1, sI1, sE1)
        return carry

    lax.fori_loop(0, 18, body, 0)
    wait_pre(ib0, rb0, sI0, sE0)
    scat(ib0, rb0)
    pre(38, ib0, rb0, sI0, sE0)
    wait_pre(ib1, rb1, sI1, sE1)
    scat(ib1, rb1)
    toff = pl.multiple_of(base + NFULL * CH, 8)
    pltpu.async_copy(idx_s_hbm.at[pl.ds(toff, TAIL)], it, sI1)
    pltpu.async_copy(emb_hbm.at[pl.ds(toff, TAIL)], rt, sE1)
    wait_pre(ib0, rb0, sI0, sE0)
    scat(ib0, rb0)
    pltpu.make_async_copy(idx_s_hbm.at[pl.ds(0, TAIL)], it, sI1).wait()
    pltpu.make_async_copy(emb_hbm.at[pl.ds(0, TAIL)], rt, sE1).wait()
    pltpu.sync_copy(rt, acc.at[it], add=True)

    plsc.subcore_barrier()
    ooff = pl.multiple_of(cid * NPAD + sid * RPW, 8)
    pltpu.sync_copy(acc.at[pl.ds(sid * RPW, RPW)], out_hbm.at[pl.ds(ooff, RPW)])


@functools.lru_cache(maxsize=None)
def _build_sc_kernels():
    mesh = plsc.VectorSubcoreMesh(core_axis_name="c", subcore_axis_name="s",
                                  num_cores=NC, num_subcores=NS)
    gather = pl.kernel(
        _sc_gather_body,
        out_type=jax.ShapeDtypeStruct((EG, H), jnp.float32),
        mesh=mesh,
        scratch_types=[
            pltpu.VMEM((EPW,), jnp.int32),
            pltpu.VMEM((EPW,), jnp.int32),
            pltpu.VMEM((CH, H), jnp.float32),
            pltpu.VMEM((CH, H), jnp.float32),
            pltpu.VMEM((TAIL, H), jnp.float32),
            pltpu.SemaphoreType.DMA,
            pltpu.SemaphoreType.DMA,
            pltpu.SemaphoreType.DMA,
            pltpu.SemaphoreType.DMA,
            pltpu.SemaphoreType.DMA,
            pltpu.SemaphoreType.DMA,
        ],
    )
    scatter = pl.kernel(
        _sc_scatter_body,
        out_type=jax.ShapeDtypeStruct((NC * NPAD, DE), jnp.float32),
        mesh=mesh,
        scratch_types=[
            pltpu.VMEM((CH,), jnp.int32),
            pltpu.VMEM((CH, DE), jnp.float32),
            pltpu.VMEM((CH,), jnp.int32),
            pltpu.VMEM((CH, DE), jnp.float32),
            pltpu.VMEM((TAIL,), jnp.int32),
            pltpu.VMEM((TAIL, DE), jnp.float32),
            pltpu.VMEM_SHARED((NPAD, DE), jnp.float32),
            pltpu.SemaphoreType.DMA,
            pltpu.SemaphoreType.DMA,
            pltpu.SemaphoreType.DMA,
            pltpu.SemaphoreType.DMA,
        ],
    )
    return gather, scatter


def _sc_gather(p, q, idx_s, idx_r):
    return _build_sc_kernels()[0](p, q, idx_s, idx_r)


def _sc_scatter(emb, idx_s, zeros):
    return _build_sc_kernels()[1](emb, idx_s, zeros)


def kernel(V, E, edges, ew0, eb0, ew1, eb1, ew2, eb2, eg, ebt,
           nw0, nb0, nw1, nb1, nw2, nb2, ng, nbt):
    v2 = V.reshape(N, DN)
    e2 = E.reshape(EG, DE).astype(jnp.bfloat16)
    idx_s = edges.reshape(EG, 2)[:, 0]
    idx_r = edges.reshape(EG, 2)[:, 1]

    w_cat = jnp.concatenate([ew0[:DN], ew0[DN:2 * DN], nw0[:DN]], axis=1)
    p, q, r = _precompute(v2, w_cat)

    pq = _sc_gather(p, q, idx_s, idx_r)

    edge_emb = _edge_mlp(
        pq, e2, ew0[2 * DN:].astype(jnp.bfloat16), eb0.reshape(1, H),
        ew1.astype(jnp.bfloat16), eb1.reshape(1, H),
        ew2.astype(jnp.bfloat16), eb2.reshape(1, DE),
        eg.reshape(1, DE), ebt.reshape(1, DE))

    zeros = jnp.zeros((RPW, DE), jnp.float32)
    partials = _sc_scatter(edge_emb, idx_s, zeros)
    p0 = partials[:N]
    p1 = partials[NPAD:NPAD + N]

    node_emb = _node_mlp(
        r, p0, p1, nw0[DN:], nb0.reshape(1, H),
        nw1, nb1.reshape(1, H), nw2, nb2.reshape(1, OUT),
        ng.reshape(1, OUT), nbt.reshape(1, OUT))

    return (node_emb.reshape(1, N, OUT), edge_emb.reshape(1, EG, DE))
